# baseline (speedup 1.0000x reference)
import functools

import jax
import jax.numpy as jnp
from jax import lax
from jax.experimental import pallas as pl
from jax.experimental.pallas import tpu as pltpu
from jax.experimental.pallas import tpu_sc as plsc

N_NODES = 10000
N_EDGES = 320000
D_IN = 128
D_HID = 128
D_OUT = 64
DW = 64

NPAD = 10240
DUMMY = N_NODES
NW = 32
CHUNK = 128
CPW = 80
NBUF = 4
EPAD = NW * CPW * CHUNK
ROWS_PER_TILE = NPAD // 16


@functools.lru_cache(maxsize=None)
def _sc_aggregate(with_deg):
    mesh = plsc.VectorSubcoreMesh(core_axis_name="c", subcore_axis_name="s",
                                  num_cores=2, num_subcores=16)
    out_t = [
        jax.ShapeDtypeStruct((NPAD, DW), jnp.float32),
        jax.ShapeDtypeStruct((NPAD, DW), jnp.float32),
    ]
    scratch = [
        pltpu.VMEM((CPW, CHUNK), jnp.int32),
        pltpu.VMEM((CPW, CHUNK), jnp.int32),
        [pltpu.VMEM((CHUNK, DW), jnp.float32) for _ in range(NBUF)],
        pltpu.VMEM_SHARED((NPAD, DW), jnp.float32),
        [pltpu.SemaphoreType.DMA for _ in range(NBUF)],
    ]
    if with_deg:
        out_t += [jax.ShapeDtypeStruct((NPAD, 16), jnp.float32),
                  jax.ShapeDtypeStruct((NPAD, 16), jnp.float32)]
        scratch += [pltpu.VMEM((CHUNK, 16), jnp.float32),
                    pltpu.VMEM_SHARED((NPAD, 16), jnp.float32)]

    def agg(*refs):
        if with_deg:
            (p_hbm, src_hbm, dst_hbm, z64_hbm, z16_hbm, ones_hbm,
             outa, outb, dega, degb,
             src_v, dst_v, rows, acc, sems, ones_v, dacc) = refs
        else:
            (p_hbm, src_hbm, dst_hbm, z64_hbm, z16_hbm, ones_hbm,
             outa, outb,
             src_v, dst_v, rows, acc, sems) = refs
        c = lax.axis_index("c")
        s = lax.axis_index("s")
        w = c * 16 + s

        pltpu.sync_copy(src_hbm.at[pl.ds(w * CPW, CPW)], src_v)
        pltpu.sync_copy(dst_hbm.at[pl.ds(w * CPW, CPW)], dst_v)
        rbase = s * ROWS_PER_TILE
        pltpu.sync_copy(z64_hbm, acc.at[pl.ds(rbase, ROWS_PER_TILE)])
        if with_deg:
            pltpu.sync_copy(ones_hbm, ones_v)
            pltpu.sync_copy(z16_hbm, dacc.at[pl.ds(rbase, ROWS_PER_TILE)])
        plsc.subcore_barrier()

        for b in range(NBUF):
            pltpu.async_copy(p_hbm.at[src_v.at[b]], rows[b], sems[b])

        def step(i, carry):
            j = i * NBUF
            for b in range(NBUF):
                jj = j + b
                pltpu.make_async_copy(p_hbm.at[src_v.at[jj]], rows[b],
                                      sems[b]).wait()
                pltpu.sync_copy(rows[b], acc.at[dst_v.at[jj]], add=True)
                if with_deg:
                    pltpu.sync_copy(ones_v, dacc.at[dst_v.at[jj]], add=True)

                @pl.when(jj + NBUF < CPW)
                def _():
                    pltpu.async_copy(p_hbm.at[src_v.at[jj + NBUF]], rows[b],
                                     sems[b])
            return carry

        lax.fori_loop(0, CPW // NBUF, step, 0)
        plsc.subcore_barrier()

        row_slice = pl.ds(rbase, ROWS_PER_TILE)

        @pl.when(c == 0)
        def _():
            pltpu.sync_copy(acc.at[row_slice], outa.at[row_slice])
            if with_deg:
                pltpu.sync_copy(dacc.at[row_slice], dega.at[row_slice])

        @pl.when(c == 1)
        def _():
            pltpu.sync_copy(acc.at[row_slice], outb.at[row_slice])
            if with_deg:
                pltpu.sync_copy(dacc.at[row_slice], degb.at[row_slice])

    return pl.kernel(
        agg, mesh=mesh, out_type=out_t, scratch_types=scratch,
        compiler_params=pltpu.CompilerParams(use_tc_tiling_on_sc=False))


BLK = 1024


def _tc_matmul(x, w):
    k, d = w.shape

    def body(x_ref, w_ref, o_ref):
        o_ref[...] = jnp.dot(x_ref[...], w_ref[...],
                             preferred_element_type=jnp.float32)

    return pl.pallas_call(
        body,
        grid=(NPAD // BLK,),
        in_specs=[pl.BlockSpec((BLK, k), lambda i: (i, 0)),
                  pl.BlockSpec((k, d), lambda i: (0, 0))],
        out_specs=pl.BlockSpec((BLK, d), lambda i: (i, 0)),
        out_shape=jax.ShapeDtypeStruct((NPAD, d), jnp.float32),
    )(x, w)


def _tc_mid(sL0, sL1, sR0, sR1, p1, dega, degb, b1, w2):
    d_in, d_out = w2.shape

    def body(sl0, sl1, sr0, sr1, p_ref, da, db, b_ref, w_ref, o_ref):
        denom = (da[...] + db[...])[:, 0:1] + 1.0
        p = p_ref[...]
        hL = (sl0[...] + sl1[...] + p[:, :DW]) / denom + b_ref[...][:, :DW]
        hR = (sr0[...] + sr1[...] + p[:, DW:]) / denom + b_ref[...][:, DW:]
        h = jnp.maximum(jnp.concatenate([hL, hR], axis=1), 0.0)
        o_ref[...] = jnp.dot(h, w_ref[...], preferred_element_type=jnp.float32)

    wide = pl.BlockSpec((BLK, DW), lambda i: (i, 0))
    return pl.pallas_call(
        body,
        grid=(NPAD // BLK,),
        in_specs=[wide, wide, wide, wide,
                  pl.BlockSpec((BLK, d_in), lambda i: (i, 0)),
                  pl.BlockSpec((BLK, 16), lambda i: (i, 0)),
                  pl.BlockSpec((BLK, 16), lambda i: (i, 0)),
                  pl.BlockSpec((1, d_in), lambda i: (0, 0)),
                  pl.BlockSpec((d_in, d_out), lambda i: (0, 0))],
        out_specs=pl.BlockSpec((BLK, d_out), lambda i: (i, 0)),
        out_shape=jax.ShapeDtypeStruct((NPAD, d_out), jnp.float32),
    )(sL0, sL1, sR0, sR1, p1, dega, degb, b1, w2)


def _tc_final(s2a, s2b, p2, dega, degb, b2):
    d = p2.shape[1]

    def body(sa_ref, sb_ref, p_ref, da_ref, db_ref, b_ref, o_ref):
        denom = (da_ref[...] + db_ref[...])[:, 0:1] + 1.0
        o_ref[...] = (sa_ref[...] + sb_ref[...] + p_ref[...]) / denom + b_ref[...]

    return pl.pallas_call(
        body,
        grid=(NPAD // BLK,),
        in_specs=[pl.BlockSpec((BLK, d), lambda i: (i, 0)),
                  pl.BlockSpec((BLK, d), lambda i: (i, 0)),
                  pl.BlockSpec((BLK, d), lambda i: (i, 0)),
                  pl.BlockSpec((BLK, 16), lambda i: (i, 0)),
                  pl.BlockSpec((BLK, 16), lambda i: (i, 0)),
                  pl.BlockSpec((1, d), lambda i: (0, 0))],
        out_specs=pl.BlockSpec((BLK, d), lambda i: (i, 0)),
        out_shape=jax.ShapeDtypeStruct((NPAD, d), jnp.float32),
    )(s2a, s2b, p2, dega, degb, b2)


def kernel(x, edge_index, W1, b1, W2, b2):
    f32 = jnp.float32
    src = edge_index[0].astype(jnp.int32)
    dst = edge_index[1].astype(jnp.int32)
    npad_e = EPAD - N_EDGES
    src_p = jnp.concatenate([src, jnp.zeros((npad_e,), jnp.int32)])
    pad_dst = DUMMY + (jnp.arange(npad_e, dtype=jnp.int32) % (NPAD - N_NODES))
    dst_p = jnp.concatenate([dst, pad_dst])
    src_p = src_p.reshape(NW * CPW, CHUNK)
    dst_p = dst_p.reshape(NW * CPW, CHUNK)
    xp = jnp.concatenate([x, jnp.zeros((NPAD - N_NODES, D_IN), f32)])
    z64 = jnp.zeros((ROWS_PER_TILE, DW), f32)
    z16 = jnp.zeros((ROWS_PER_TILE, 16), f32)
    ones16 = jnp.ones((CHUNK, 16), f32)
    b1r = b1.reshape(1, D_HID)
    b2r = b2.reshape(1, D_OUT)

    p1 = _tc_matmul(xp, W1)
    p1L = p1[:, :DW]
    p1R = p1[:, DW:]
    sL0, sL1, dega, degb = _sc_aggregate(True)(
        p1L, src_p, dst_p, z64, z16, ones16)
    sR0, sR1 = _sc_aggregate(False)(p1R, src_p, dst_p, z64, z16, ones16)
    p2 = _tc_mid(sL0, sL1, sR0, sR1, p1, dega, degb, b1r, W2)

    s2a, s2b = _sc_aggregate(False)(p2, src_p, dst_p, z64, z16, ones16)
    out = _tc_final(s2a, s2b, p2, dega, degb, b2r)
    return out[:N_NODES]


# speedup vs baseline: 1.0660x; 1.0660x over previous
import functools

import jax
import jax.numpy as jnp
from jax import lax
from jax.experimental import pallas as pl
from jax.experimental.pallas import tpu as pltpu
from jax.experimental.pallas import tpu_sc as plsc

N_NODES = 10000
N_EDGES = 320000
D_IN = 128
D_HID = 128
D_OUT = 64
DW = 64

NPAD = 10240
DUMMY = N_NODES
NW = 32
CHUNK = 128
CPW = 80
NBUF = 4
EPAD = NW * CPW * CHUNK
ROWS_PER_TILE = NPAD // 16


@functools.lru_cache(maxsize=None)
def _sc_aggregate(with_deg):
    mesh = plsc.VectorSubcoreMesh(core_axis_name="c", subcore_axis_name="s",
                                  num_cores=2, num_subcores=16)
    out_t = [
        jax.ShapeDtypeStruct((NPAD, DW), jnp.float32),
        jax.ShapeDtypeStruct((NPAD, DW), jnp.float32),
    ]
    scratch = [
        pltpu.VMEM((CPW, CHUNK), jnp.int32),
        pltpu.VMEM((CPW, CHUNK), jnp.int32),
        [pltpu.VMEM((CHUNK, DW), jnp.float32) for _ in range(NBUF)],
        pltpu.VMEM_SHARED((NPAD, DW), jnp.float32),
        [pltpu.SemaphoreType.DMA for _ in range(NBUF)],
    ]
    if with_deg:
        out_t += [jax.ShapeDtypeStruct((NPAD, 16), jnp.float32),
                  jax.ShapeDtypeStruct((NPAD, 16), jnp.float32)]
        scratch += [pltpu.VMEM((CHUNK, 16), jnp.float32),
                    pltpu.VMEM_SHARED((NPAD, 16), jnp.float32)]

    def agg(*refs):
        if with_deg:
            (p_hbm, src_hbm, srchi_hbm, dst_hbm, z64_hbm, z16_hbm, ones_hbm,
             outa, outb, dega, degb,
             src_v, dst_v, rows, acc, sems, ones_v, dacc) = refs
        else:
            (p_hbm, src_hbm, srchi_hbm, dst_hbm, z64_hbm, z16_hbm, ones_hbm,
             outa, outb,
             src_v, dst_v, rows, acc, sems) = refs
        c = lax.axis_index("c")
        s = lax.axis_index("s")
        w = c * 16 + s

        @pl.when(c == 0)
        def _():
            pltpu.sync_copy(src_hbm.at[pl.ds(w * CPW, CPW)], src_v)

        @pl.when(c == 1)
        def _():
            pltpu.sync_copy(srchi_hbm.at[pl.ds(w * CPW, CPW)], src_v)

        pltpu.sync_copy(dst_hbm.at[pl.ds(w * CPW, CPW)], dst_v)
        rbase = s * ROWS_PER_TILE
        pltpu.sync_copy(z64_hbm, acc.at[pl.ds(rbase, ROWS_PER_TILE)])
        if with_deg:
            pltpu.sync_copy(ones_hbm, ones_v)
            pltpu.sync_copy(z16_hbm, dacc.at[pl.ds(rbase, ROWS_PER_TILE)])
        plsc.subcore_barrier()

        for b in range(NBUF):
            pltpu.async_copy(p_hbm.at[src_v.at[b]], rows[b], sems[b])

        def step(i, carry):
            j = i * NBUF
            for b in range(NBUF):
                jj = j + b
                pltpu.make_async_copy(p_hbm.at[src_v.at[jj]], rows[b],
                                      sems[b]).wait()
                pltpu.sync_copy(rows[b], acc.at[dst_v.at[jj]], add=True)
                if with_deg:
                    pltpu.sync_copy(ones_v, dacc.at[dst_v.at[jj]], add=True)

                @pl.when(jj + NBUF < CPW)
                def _():
                    pltpu.async_copy(p_hbm.at[src_v.at[jj + NBUF]], rows[b],
                                     sems[b])
            return carry

        lax.fori_loop(0, CPW // NBUF, step, 0)
        plsc.subcore_barrier()

        row_slice = pl.ds(rbase, ROWS_PER_TILE)

        @pl.when(c == 0)
        def _():
            pltpu.sync_copy(acc.at[row_slice], outa.at[row_slice])
            if with_deg:
                pltpu.sync_copy(dacc.at[row_slice], dega.at[row_slice])

        @pl.when(c == 1)
        def _():
            pltpu.sync_copy(acc.at[row_slice], outb.at[row_slice])
            if with_deg:
                pltpu.sync_copy(dacc.at[row_slice], degb.at[row_slice])

    return pl.kernel(
        agg, mesh=mesh, out_type=out_t, scratch_types=scratch,
        compiler_params=pltpu.CompilerParams(use_tc_tiling_on_sc=False))


BLK = 1024


def _tc_matmul(x, w):
    k, d = w.shape

    def body(x_ref, w_ref, o_ref):
        o_ref[...] = jnp.dot(x_ref[...], w_ref[...],
                             preferred_element_type=jnp.float32)

    return pl.pallas_call(
        body,
        grid=(NPAD // BLK,),
        in_specs=[pl.BlockSpec((BLK, k), lambda i: (i, 0)),
                  pl.BlockSpec((k, d), lambda i: (0, 0))],
        out_specs=pl.BlockSpec((BLK, d), lambda i: (i, 0)),
        out_shape=jax.ShapeDtypeStruct((NPAD, d), jnp.float32),
    )(x, w)


def _tc_mid(sL0, sL1, sR0, sR1, p1, dega, degb, b1, w2):
    d_in, d_out = w2.shape

    def body(sl0, sl1, sr0, sr1, p_ref, da, db, b_ref, w_ref, o_ref):
        denom = (da[...] + db[...])[:, 0:1] + 1.0
        p = p_ref[...]
        hL = (sl0[...] + sl1[...] + p[:, :DW]) / denom + b_ref[...][:, :DW]
        hR = (sr0[...] + sr1[...] + p[:, DW:]) / denom + b_ref[...][:, DW:]
        h = jnp.maximum(jnp.concatenate([hL, hR], axis=1), 0.0)
        o_ref[...] = jnp.dot(h, w_ref[...], preferred_element_type=jnp.float32)

    wide = pl.BlockSpec((BLK, DW), lambda i: (i, 0))
    return pl.pallas_call(
        body,
        grid=(NPAD // BLK,),
        in_specs=[wide, wide, wide, wide,
                  pl.BlockSpec((BLK, d_in), lambda i: (i, 0)),
                  pl.BlockSpec((BLK, 16), lambda i: (i, 0)),
                  pl.BlockSpec((BLK, 16), lambda i: (i, 0)),
                  pl.BlockSpec((1, d_in), lambda i: (0, 0)),
                  pl.BlockSpec((d_in, d_out), lambda i: (0, 0))],
        out_specs=pl.BlockSpec((BLK, d_out), lambda i: (i, 0)),
        out_shape=jax.ShapeDtypeStruct((NPAD, d_out), jnp.float32),
    )(sL0, sL1, sR0, sR1, p1, dega, degb, b1, w2)


def _tc_final(s2a, s2b, p2, dega, degb, b2):
    d = p2.shape[1]

    def body(sa_ref, sb_ref, p_ref, da_ref, db_ref, b_ref, o_ref):
        denom = (da_ref[...] + db_ref[...])[:, 0:1] + 1.0
        o_ref[...] = (sa_ref[...] + sb_ref[...] + p_ref[...]) / denom + b_ref[...]

    return pl.pallas_call(
        body,
        grid=(NPAD // BLK,),
        in_specs=[pl.BlockSpec((BLK, d), lambda i: (i, 0)),
                  pl.BlockSpec((BLK, d), lambda i: (i, 0)),
                  pl.BlockSpec((BLK, d), lambda i: (i, 0)),
                  pl.BlockSpec((BLK, 16), lambda i: (i, 0)),
                  pl.BlockSpec((BLK, 16), lambda i: (i, 0)),
                  pl.BlockSpec((1, d), lambda i: (0, 0))],
        out_specs=pl.BlockSpec((BLK, d), lambda i: (i, 0)),
        out_shape=jax.ShapeDtypeStruct((NPAD, d), jnp.float32),
    )(s2a, s2b, p2, dega, degb, b2)


def kernel(x, edge_index, W1, b1, W2, b2):
    f32 = jnp.float32
    src = edge_index[0].astype(jnp.int32)
    dst = edge_index[1].astype(jnp.int32)
    npad_e = EPAD - N_EDGES
    src_p = jnp.concatenate([src, jnp.zeros((npad_e,), jnp.int32)])
    pad_dst = DUMMY + (jnp.arange(npad_e, dtype=jnp.int32) % (NPAD - N_NODES))
    dst_p = jnp.concatenate([dst, pad_dst])
    src_p = src_p.reshape(NW * CPW, CHUNK)
    dst_p = dst_p.reshape(NW * CPW, CHUNK)
    xp = jnp.concatenate([x, jnp.zeros((NPAD - N_NODES, D_IN), f32)])
    z64 = jnp.zeros((ROWS_PER_TILE, DW), f32)
    z16 = jnp.zeros((ROWS_PER_TILE, 16), f32)
    ones16 = jnp.ones((CHUNK, 16), f32)
    b1r = b1.reshape(1, D_HID)
    b2r = b2.reshape(1, D_OUT)

    src_hi = src_p + NPAD

    def doubled(p):
        return jnp.concatenate([p, p])

    p1 = _tc_matmul(xp, W1)
    p1L = doubled(p1[:, :DW])
    p1R = doubled(p1[:, DW:])
    sL0, sL1, dega, degb = _sc_aggregate(True)(
        p1L, src_p, src_hi, dst_p, z64, z16, ones16)
    sR0, sR1 = _sc_aggregate(False)(p1R, src_p, src_hi, dst_p, z64, z16, ones16)
    p2 = _tc_mid(sL0, sL1, sR0, sR1, p1, dega, degb, b1r, W2)

    s2a, s2b = _sc_aggregate(False)(doubled(p2), src_p, src_hi, dst_p,
                                    z64, z16, ones16)
    out = _tc_final(s2a, s2b, p2, dega, degb, b2r)
    return out[:N_NODES]


# speedup vs baseline: 2.1869x; 2.0514x over previous
import functools

import jax
import jax.numpy as jnp
from jax import lax
from jax.experimental import pallas as pl
from jax.experimental.pallas import tpu as pltpu
from jax.experimental.pallas import tpu_sc as plsc

N_NODES = 10000
N_EDGES = 320000
D_IN = 128
D_HID = 128
D_OUT = 64
DW = 32

NPAD = 10240
DUMMY = N_NODES
NW = 32
CHUNK = 128
CPW = 80
NBUF = 4
EPAD = NW * CPW * CHUNK
ROWS_PER_TILE = NPAD // 16


@functools.lru_cache(maxsize=None)
def _sc_aggregate(with_deg):
    mesh = plsc.VectorSubcoreMesh(core_axis_name="c", subcore_axis_name="s",
                                  num_cores=2, num_subcores=16)
    out_t = [
        jax.ShapeDtypeStruct((NPAD, DW), jnp.float32),
        jax.ShapeDtypeStruct((NPAD, DW), jnp.float32),
    ]
    scratch = [
        pltpu.VMEM((CPW, CHUNK), jnp.int32),
        pltpu.VMEM((CPW, CHUNK), jnp.int32),
        [pltpu.VMEM((CHUNK, DW), jnp.float32) for _ in range(NBUF)],
        pltpu.VMEM_SHARED((NPAD, DW), jnp.float32),
        pltpu.VMEM_SHARED((NPAD, DW), jnp.float32),
        [pltpu.SemaphoreType.DMA for _ in range(NBUF)],
    ]
    if with_deg:
        out_t += [jax.ShapeDtypeStruct((NPAD, 16), jnp.float32),
                  jax.ShapeDtypeStruct((NPAD, 16), jnp.float32)]
        scratch += [pltpu.VMEM((CHUNK, 16), jnp.float32),
                    pltpu.VMEM_SHARED((NPAD, 16), jnp.float32)]

    def agg(*refs):
        if with_deg:
            (p_hbm, src_hbm, dst_hbm, zd_hbm, z16_hbm, ones_hbm,
             outa, outb, dega, degb,
             src_v, dst_v, rows, tab, acc, sems, ones_v, dacc) = refs
        else:
            (p_hbm, src_hbm, dst_hbm, zd_hbm, z16_hbm, ones_hbm,
             outa, outb,
             src_v, dst_v, rows, tab, acc, sems) = refs
        c = lax.axis_index("c")
        s = lax.axis_index("s")
        w = c * 16 + s

        rbase = s * ROWS_PER_TILE
        row_slice = pl.ds(rbase, ROWS_PER_TILE)
        pltpu.sync_copy(src_hbm.at[pl.ds(w * CPW, CPW)], src_v)
        pltpu.sync_copy(dst_hbm.at[pl.ds(w * CPW, CPW)], dst_v)
        pltpu.sync_copy(p_hbm.at[row_slice], tab.at[row_slice])
        pltpu.sync_copy(zd_hbm, acc.at[row_slice])
        if with_deg:
            pltpu.sync_copy(ones_hbm, ones_v)
            pltpu.sync_copy(z16_hbm, dacc.at[row_slice])
        plsc.subcore_barrier()

        for b in range(NBUF):
            pltpu.async_copy(tab.at[src_v.at[b]], rows[b], sems[b])

        def step(i, carry):
            j = i * NBUF
            for b in range(NBUF):
                jj = j + b
                pltpu.make_async_copy(tab.at[src_v.at[jj]], rows[b],
                                      sems[b]).wait()
                pltpu.sync_copy(rows[b], acc.at[dst_v.at[jj]], add=True)
                if with_deg:
                    pltpu.sync_copy(ones_v, dacc.at[dst_v.at[jj]], add=True)

                @pl.when(jj + NBUF < CPW)
                def _():
                    pltpu.async_copy(tab.at[src_v.at[jj + NBUF]], rows[b],
                                     sems[b])
            return carry

        lax.fori_loop(0, CPW // NBUF, step, 0)
        plsc.subcore_barrier()

        @pl.when(c == 0)
        def _():
            pltpu.sync_copy(acc.at[row_slice], outa.at[row_slice])
            if with_deg:
                pltpu.sync_copy(dacc.at[row_slice], dega.at[row_slice])

        @pl.when(c == 1)
        def _():
            pltpu.sync_copy(acc.at[row_slice], outb.at[row_slice])
            if with_deg:
                pltpu.sync_copy(dacc.at[row_slice], degb.at[row_slice])

    return pl.kernel(
        agg, mesh=mesh, out_type=out_t, scratch_types=scratch,
        compiler_params=pltpu.CompilerParams(use_tc_tiling_on_sc=False))


BLK = 1024


def _tc_matmul(x, w):
    k, d = w.shape

    def body(x_ref, w_ref, o_ref):
        o_ref[...] = jnp.dot(x_ref[...], w_ref[...],
                             preferred_element_type=jnp.float32)

    return pl.pallas_call(
        body,
        grid=(NPAD // BLK,),
        in_specs=[pl.BlockSpec((BLK, k), lambda i: (i, 0)),
                  pl.BlockSpec((k, d), lambda i: (0, 0))],
        out_specs=pl.BlockSpec((BLK, d), lambda i: (i, 0)),
        out_shape=jax.ShapeDtypeStruct((NPAD, d), jnp.float32),
    )(x, w)


def _tc_mid(parts, p1, dega, degb, b1, w2):
    d_in, d_out = w2.shape

    def body(*refs):
        (s0a, s0b, s1a, s1b, s2a, s2b, s3a, s3b,
         p_ref, da, db, b_ref, w_ref, o_ref) = refs
        denom = (da[...] + db[...])[:, 0:1] + 1.0
        p = p_ref[...]
        bb = b_ref[...]
        blocks = []
        for k, (sa, sb) in enumerate(((s0a, s0b), (s1a, s1b),
                                      (s2a, s2b), (s3a, s3b))):
            pk = p[:, k * DW:(k + 1) * DW]
            bk = bb[:, k * DW:(k + 1) * DW]
            blocks.append((sa[...] + sb[...] + pk) / denom + bk)
        h = jnp.maximum(jnp.concatenate(blocks, axis=1), 0.0)
        o_ref[...] = jnp.dot(h, w_ref[...], preferred_element_type=jnp.float32)

    nar = pl.BlockSpec((BLK, DW), lambda i: (i, 0))
    return pl.pallas_call(
        body,
        grid=(NPAD // BLK,),
        in_specs=[nar] * 8 + [
            pl.BlockSpec((BLK, d_in), lambda i: (i, 0)),
            pl.BlockSpec((BLK, 16), lambda i: (i, 0)),
            pl.BlockSpec((BLK, 16), lambda i: (i, 0)),
            pl.BlockSpec((1, d_in), lambda i: (0, 0)),
            pl.BlockSpec((d_in, d_out), lambda i: (0, 0))],
        out_specs=pl.BlockSpec((BLK, d_out), lambda i: (i, 0)),
        out_shape=jax.ShapeDtypeStruct((NPAD, d_out), jnp.float32),
    )(*parts, p1, dega, degb, b1, w2)


def _tc_final(parts, p2, dega, degb, b2):
    d = p2.shape[1]

    def body(*refs):
        s0a, s0b, s1a, s1b, p_ref, da, db, b_ref, o_ref = refs
        denom = (da[...] + db[...])[:, 0:1] + 1.0
        p = p_ref[...]
        bb = b_ref[...]
        blocks = []
        for k, (sa, sb) in enumerate(((s0a, s0b), (s1a, s1b))):
            pk = p[:, k * DW:(k + 1) * DW]
            bk = bb[:, k * DW:(k + 1) * DW]
            blocks.append((sa[...] + sb[...] + pk) / denom + bk)
        o_ref[...] = jnp.concatenate(blocks, axis=1)

    nar = pl.BlockSpec((BLK, DW), lambda i: (i, 0))
    return pl.pallas_call(
        body,
        grid=(NPAD // BLK,),
        in_specs=[nar] * 4 + [
            pl.BlockSpec((BLK, d), lambda i: (i, 0)),
            pl.BlockSpec((BLK, 16), lambda i: (i, 0)),
            pl.BlockSpec((BLK, 16), lambda i: (i, 0)),
            pl.BlockSpec((1, d), lambda i: (0, 0))],
        out_specs=pl.BlockSpec((BLK, d), lambda i: (i, 0)),
        out_shape=jax.ShapeDtypeStruct((NPAD, d), jnp.float32),
    )(*parts, p2, dega, degb, b2)


def kernel(x, edge_index, W1, b1, W2, b2):
    f32 = jnp.float32
    src = edge_index[0].astype(jnp.int32)
    dst = edge_index[1].astype(jnp.int32)
    npad_e = EPAD - N_EDGES
    src_p = jnp.concatenate([src, jnp.zeros((npad_e,), jnp.int32)])
    pad_dst = DUMMY + (jnp.arange(npad_e, dtype=jnp.int32) % (NPAD - N_NODES))
    dst_p = jnp.concatenate([dst, pad_dst])
    src_p = src_p.reshape(NW * CPW, CHUNK)
    dst_p = dst_p.reshape(NW * CPW, CHUNK)
    xp = jnp.concatenate([x, jnp.zeros((NPAD - N_NODES, D_IN), f32)])
    zd = jnp.zeros((ROWS_PER_TILE, DW), f32)
    z16 = jnp.zeros((ROWS_PER_TILE, 16), f32)
    ones16 = jnp.ones((CHUNK, 16), f32)
    b1r = b1.reshape(1, D_HID)
    b2r = b2.reshape(1, D_OUT)

    p1 = _tc_matmul(xp, W1)
    parts1 = []
    dega = degb = None
    for k in range(D_HID // DW):
        blk = p1[:, k * DW:(k + 1) * DW]
        if k == 0:
            a, b, dega, degb = _sc_aggregate(True)(
                blk, src_p, dst_p, zd, z16, ones16)
        else:
            a, b = _sc_aggregate(False)(blk, src_p, dst_p, zd, z16, ones16)
        parts1 += [a, b]
    p2 = _tc_mid(parts1, p1, dega, degb, b1r, W2)

    parts2 = []
    for k in range(D_OUT // DW):
        blk = p2[:, k * DW:(k + 1) * DW]
        a, b = _sc_aggregate(False)(blk, src_p, dst_p, zd, z16, ones16)
        parts2 += [a, b]
    out = _tc_final(parts2, p2, dega, degb, b2r)
    return out[:N_NODES]


# speedup vs baseline: 2.4259x; 1.1093x over previous
import functools

import jax
import jax.numpy as jnp
from jax import lax
from jax.experimental import pallas as pl
from jax.experimental.pallas import tpu as pltpu
from jax.experimental.pallas import tpu_sc as plsc

N_NODES = 10000
N_EDGES = 320000
D_IN = 128
D_HID = 128
D_OUT = 64
DW = 32

NPAD = 10240
DUMMY = N_NODES
NT = 16
CHUNK = 128
CPW = 160
NBUF = 4
EPAD = NT * CPW * CHUNK
ROWS_PER_TILE = NPAD // NT


@functools.lru_cache(maxsize=None)
def _sc_layer(npairs, with_deg):
    nblk = 2 * npairs
    mesh = plsc.VectorSubcoreMesh(core_axis_name="c", subcore_axis_name="s",
                                  num_cores=2, num_subcores=16)
    out_t = [jax.ShapeDtypeStruct((NPAD, DW), jnp.float32)] * nblk
    scratch = [
        pltpu.VMEM((CPW, CHUNK), jnp.int32),
        pltpu.VMEM((CPW, CHUNK), jnp.int32),
        [pltpu.VMEM((CHUNK, DW), jnp.float32) for _ in range(NBUF)],
        pltpu.VMEM_SHARED((NPAD, DW), jnp.float32),
        pltpu.VMEM_SHARED((NPAD, DW), jnp.float32),
        [pltpu.SemaphoreType.DMA for _ in range(NBUF)],
    ]
    if with_deg:
        out_t += [jax.ShapeDtypeStruct((NPAD, 16), jnp.float32)]
        scratch += [pltpu.VMEM((CHUNK, 16), jnp.float32),
                    pltpu.VMEM_SHARED((NPAD, 16), jnp.float32)]

    def agg(*refs):
        p_blocks = refs[:nblk]
        src_hbm, dst_hbm, zd_hbm, z16_hbm, ones_hbm = refs[nblk:nblk + 5]
        outs = refs[nblk + 5:2 * nblk + 5]
        rest = refs[2 * nblk + 5:]
        if with_deg:
            deg_out = rest[0]
            (src_v, dst_v, rows, tab, acc, sems, ones_v, dacc) = rest[1:]
        else:
            (src_v, dst_v, rows, tab, acc, sems) = rest
        c = lax.axis_index("c")
        s = lax.axis_index("s")
        rbase = s * ROWS_PER_TILE
        row_slice = pl.ds(rbase, ROWS_PER_TILE)

        pltpu.sync_copy(src_hbm.at[pl.ds(s * CPW, CPW)], src_v)
        pltpu.sync_copy(dst_hbm.at[pl.ds(s * CPW, CPW)], dst_v)
        if with_deg:
            pltpu.sync_copy(ones_hbm, ones_v)
            pltpu.sync_copy(z16_hbm, dacc.at[row_slice])

        for t in range(npairs):
            @pl.when(c == 0)
            def _():
                pltpu.sync_copy(p_blocks[2 * t].at[row_slice],
                                tab.at[row_slice])

            @pl.when(c == 1)
            def _():
                pltpu.sync_copy(p_blocks[2 * t + 1].at[row_slice],
                                tab.at[row_slice])

            pltpu.sync_copy(zd_hbm, acc.at[row_slice])
            plsc.subcore_barrier()

            for b in range(NBUF):
                pltpu.async_copy(tab.at[src_v.at[b]], rows[b], sems[b])

            count_deg = with_deg and t == 0

            def step(i, carry):
                j = i * NBUF
                for b in range(NBUF):
                    jj = j + b
                    pltpu.make_async_copy(tab.at[src_v.at[jj]], rows[b],
                                          sems[b]).wait()
                    pltpu.sync_copy(rows[b], acc.at[dst_v.at[jj]], add=True)
                    if count_deg:
                        pltpu.sync_copy(ones_v, dacc.at[dst_v.at[jj]],
                                        add=True)

                    @pl.when(jj + NBUF < CPW)
                    def _():
                        pltpu.async_copy(tab.at[src_v.at[jj + NBUF]],
                                         rows[b], sems[b])
                return carry

            lax.fori_loop(0, CPW // NBUF, step, 0)
            plsc.subcore_barrier()

            @pl.when(c == 0)
            def _():
                pltpu.sync_copy(acc.at[row_slice], outs[2 * t].at[row_slice])

            @pl.when(c == 1)
            def _():
                pltpu.sync_copy(acc.at[row_slice],
                                outs[2 * t + 1].at[row_slice])

        if with_deg:
            @pl.when(c == 0)
            def _():
                pltpu.sync_copy(dacc.at[row_slice], deg_out.at[row_slice])

    return pl.kernel(
        agg, mesh=mesh, out_type=out_t, scratch_types=scratch,
        compiler_params=pltpu.CompilerParams(use_tc_tiling_on_sc=False))


BLK = 1024


def _tc_matmul_blocks(x, w):
    k, d = w.shape
    nblk = d // DW

    def body(*refs):
        x_ref, w_ref = refs[:2]
        p = jnp.dot(x_ref[...], w_ref[...], preferred_element_type=jnp.float32)
        for i, o_ref in enumerate(refs[2:]):
            o_ref[...] = p[:, i * DW:(i + 1) * DW]

    return pl.pallas_call(
        body,
        grid=(NPAD // BLK,),
        in_specs=[pl.BlockSpec((BLK, k), lambda i: (i, 0)),
                  pl.BlockSpec((k, d), lambda i: (0, 0))],
        out_specs=[pl.BlockSpec((BLK, DW), lambda i: (i, 0))] * nblk,
        out_shape=[jax.ShapeDtypeStruct((NPAD, DW), jnp.float32)] * nblk,
    )(x, w)


def _tc_mid(aggs, pblks, deg, b1, w2):
    d_in, d_out = w2.shape
    nin = d_in // DW
    nout = d_out // DW

    def body(*refs):
        ins = refs[:nin]
        ps = refs[nin:2 * nin]
        dg, b_ref, w_ref = refs[2 * nin:2 * nin + 3]
        outs = refs[2 * nin + 3:]
        denom = dg[...][:, 0:1] + 1.0
        bb = b_ref[...]
        blocks = [(sa[...] + pk[...]) / denom + bb[:, k * DW:(k + 1) * DW]
                  for k, (sa, pk) in enumerate(zip(ins, ps))]
        h = jnp.maximum(jnp.concatenate(blocks, axis=1), 0.0)
        p2 = jnp.dot(h, w_ref[...], preferred_element_type=jnp.float32)
        for k, o_ref in enumerate(outs):
            o_ref[...] = p2[:, k * DW:(k + 1) * DW]

    nar = pl.BlockSpec((BLK, DW), lambda i: (i, 0))
    return pl.pallas_call(
        body,
        grid=(NPAD // BLK,),
        in_specs=[nar] * (2 * nin) + [
            pl.BlockSpec((BLK, 16), lambda i: (i, 0)),
            pl.BlockSpec((1, d_in), lambda i: (0, 0)),
            pl.BlockSpec((d_in, d_out), lambda i: (0, 0))],
        out_specs=[nar] * nout,
        out_shape=[jax.ShapeDtypeStruct((NPAD, DW), jnp.float32)] * nout,
    )(*aggs, *pblks, deg, b1, w2)


def _tc_final(aggs, pblks, deg, b2):
    d = D_OUT
    nin = d // DW

    def body(*refs):
        ins = refs[:nin]
        ps = refs[nin:2 * nin]
        dg, b_ref, o_ref = refs[2 * nin:]
        denom = dg[...][:, 0:1] + 1.0
        bb = b_ref[...]
        blocks = [(sa[...] + pk[...]) / denom + bb[:, k * DW:(k + 1) * DW]
                  for k, (sa, pk) in enumerate(zip(ins, ps))]
        o_ref[...] = jnp.concatenate(blocks, axis=1)

    nar = pl.BlockSpec((BLK, DW), lambda i: (i, 0))
    return pl.pallas_call(
        body,
        grid=(NPAD // BLK,),
        in_specs=[nar] * (2 * nin) + [
            pl.BlockSpec((BLK, 16), lambda i: (i, 0)),
            pl.BlockSpec((1, d), lambda i: (0, 0))],
        out_specs=pl.BlockSpec((BLK, d), lambda i: (i, 0)),
        out_shape=jax.ShapeDtypeStruct((NPAD, d), jnp.float32),
    )(*aggs, *pblks, deg, b2)


def kernel(x, edge_index, W1, b1, W2, b2):
    f32 = jnp.float32
    src = edge_index[0].astype(jnp.int32)
    dst = edge_index[1].astype(jnp.int32)
    npad_e = EPAD - N_EDGES
    src_p = jnp.concatenate([src, jnp.zeros((npad_e,), jnp.int32)])
    pad_dst = DUMMY + (jnp.arange(npad_e, dtype=jnp.int32) % (NPAD - N_NODES))
    dst_p = jnp.concatenate([dst, pad_dst])
    src_p = src_p.reshape(NT * CPW, CHUNK)
    dst_p = dst_p.reshape(NT * CPW, CHUNK)
    xp = jnp.concatenate([x, jnp.zeros((NPAD - N_NODES, D_IN), f32)])
    zd = jnp.zeros((ROWS_PER_TILE, DW), f32)
    z16 = jnp.zeros((ROWS_PER_TILE, 16), f32)
    ones16 = jnp.ones((CHUNK, 16), f32)
    b1r = b1.reshape(1, D_HID)
    b2r = b2.reshape(1, D_OUT)

    p1b = _tc_matmul_blocks(xp, W1)
    *agg1, deg = _sc_layer(2, True)(*p1b, src_p, dst_p, zd, z16, ones16)
    p2b = _tc_mid(agg1, p1b, deg, b1r, W2)

    agg2 = _sc_layer(1, False)(*p2b, src_p, dst_p, zd, z16, ones16)
    out = _tc_final(agg2, p2b, deg, b2r)
    return out[:N_NODES]


# speedup vs baseline: 2.4289x; 1.0012x over previous
import functools

import jax
import jax.numpy as jnp
from jax import lax
from jax.experimental import pallas as pl
from jax.experimental.pallas import tpu as pltpu
from jax.experimental.pallas import tpu_sc as plsc

N_NODES = 10000
N_EDGES = 320000
D_IN = 128
D_HID = 128
D_OUT = 64
DW = 32

NPAD = 10240
DUMMY = N_NODES
NT = 16
CHUNK = 128
CPW = 160
NBUF = 8
EPAD = NT * CPW * CHUNK
ROWS_PER_TILE = NPAD // NT


@functools.lru_cache(maxsize=None)
def _sc_layer(npairs, with_deg):
    nblk = 2 * npairs
    mesh = plsc.VectorSubcoreMesh(core_axis_name="c", subcore_axis_name="s",
                                  num_cores=2, num_subcores=16)
    out_t = [jax.ShapeDtypeStruct((NPAD, DW), jnp.float32)] * nblk
    scratch = [
        pltpu.VMEM((CPW, CHUNK), jnp.int32),
        pltpu.VMEM((CPW, CHUNK), jnp.int32),
        [pltpu.VMEM((CHUNK, DW), jnp.float32) for _ in range(NBUF)],
        pltpu.VMEM_SHARED((NPAD, DW), jnp.float32),
        pltpu.VMEM_SHARED((NPAD, DW), jnp.float32),
        [pltpu.SemaphoreType.DMA for _ in range(NBUF)],
    ]
    if with_deg:
        out_t += [jax.ShapeDtypeStruct((NPAD, 16), jnp.float32)]
        scratch += [pltpu.VMEM((CHUNK, 16), jnp.float32),
                    pltpu.VMEM_SHARED((NPAD, 16), jnp.float32)]

    def agg(*refs):
        p_blocks = refs[:nblk]
        src_hbm, dst_hbm, zd_hbm, z16_hbm, ones_hbm = refs[nblk:nblk + 5]
        outs = refs[nblk + 5:2 * nblk + 5]
        rest = refs[2 * nblk + 5:]
        if with_deg:
            deg_out = rest[0]
            (src_v, dst_v, rows, tab, acc, sems, ones_v, dacc) = rest[1:]
        else:
            (src_v, dst_v, rows, tab, acc, sems) = rest
        c = lax.axis_index("c")
        s = lax.axis_index("s")
        rbase = s * ROWS_PER_TILE
        row_slice = pl.ds(rbase, ROWS_PER_TILE)

        pltpu.sync_copy(src_hbm.at[pl.ds(s * CPW, CPW)], src_v)
        pltpu.sync_copy(dst_hbm.at[pl.ds(s * CPW, CPW)], dst_v)
        if with_deg:
            pltpu.sync_copy(ones_hbm, ones_v)
            pltpu.sync_copy(z16_hbm, dacc.at[row_slice])

        for t in range(npairs):
            @pl.when(c == 0)
            def _():
                pltpu.sync_copy(p_blocks[2 * t].at[row_slice],
                                tab.at[row_slice])

            @pl.when(c == 1)
            def _():
                pltpu.sync_copy(p_blocks[2 * t + 1].at[row_slice],
                                tab.at[row_slice])

            pltpu.sync_copy(zd_hbm, acc.at[row_slice])
            plsc.subcore_barrier()

            for b in range(NBUF):
                pltpu.async_copy(tab.at[src_v.at[b]], rows[b], sems[b])

            count_deg = with_deg and t == 0

            def step(i, carry):
                j = i * NBUF
                for b in range(NBUF):
                    jj = j + b
                    pltpu.make_async_copy(tab.at[src_v.at[jj]], rows[b],
                                          sems[b]).wait()
                    pltpu.sync_copy(rows[b], acc.at[dst_v.at[jj]], add=True)
                    if count_deg:
                        pltpu.sync_copy(ones_v, dacc.at[dst_v.at[jj]],
                                        add=True)

                    @pl.when(jj + NBUF < CPW)
                    def _():
                        pltpu.async_copy(tab.at[src_v.at[jj + NBUF]],
                                         rows[b], sems[b])
                return carry

            lax.fori_loop(0, CPW // NBUF, step, 0)
            plsc.subcore_barrier()

            @pl.when(c == 0)
            def _():
                pltpu.sync_copy(acc.at[row_slice], outs[2 * t].at[row_slice])

            @pl.when(c == 1)
            def _():
                pltpu.sync_copy(acc.at[row_slice],
                                outs[2 * t + 1].at[row_slice])

        if with_deg:
            @pl.when(c == 0)
            def _():
                pltpu.sync_copy(dacc.at[row_slice], deg_out.at[row_slice])

    return pl.kernel(
        agg, mesh=mesh, out_type=out_t, scratch_types=scratch,
        compiler_params=pltpu.CompilerParams(use_tc_tiling_on_sc=False))


BLK = 1024


def _tc_matmul_blocks(x, w):
    k, d = w.shape
    nblk = d // DW

    def body(*refs):
        x_ref, w_ref = refs[:2]
        p = jnp.dot(x_ref[...], w_ref[...], preferred_element_type=jnp.float32)
        for i, o_ref in enumerate(refs[2:]):
            o_ref[...] = p[:, i * DW:(i + 1) * DW]

    return pl.pallas_call(
        body,
        grid=(NPAD // BLK,),
        in_specs=[pl.BlockSpec((BLK, k), lambda i: (i, 0)),
                  pl.BlockSpec((k, d), lambda i: (0, 0))],
        out_specs=[pl.BlockSpec((BLK, DW), lambda i: (i, 0))] * nblk,
        out_shape=[jax.ShapeDtypeStruct((NPAD, DW), jnp.float32)] * nblk,
    )(x, w)


def _tc_mid(aggs, pblks, deg, b1, w2):
    d_in, d_out = w2.shape
    nin = d_in // DW
    nout = d_out // DW

    def body(*refs):
        ins = refs[:nin]
        ps = refs[nin:2 * nin]
        dg, b_ref, w_ref = refs[2 * nin:2 * nin + 3]
        outs = refs[2 * nin + 3:]
        denom = dg[...][:, 0:1] + 1.0
        bb = b_ref[...]
        blocks = [(sa[...] + pk[...]) / denom + bb[:, k * DW:(k + 1) * DW]
                  for k, (sa, pk) in enumerate(zip(ins, ps))]
        h = jnp.maximum(jnp.concatenate(blocks, axis=1), 0.0)
        p2 = jnp.dot(h, w_ref[...], preferred_element_type=jnp.float32)
        for k, o_ref in enumerate(outs):
            o_ref[...] = p2[:, k * DW:(k + 1) * DW]

    nar = pl.BlockSpec((BLK, DW), lambda i: (i, 0))
    return pl.pallas_call(
        body,
        grid=(NPAD // BLK,),
        in_specs=[nar] * (2 * nin) + [
            pl.BlockSpec((BLK, 16), lambda i: (i, 0)),
            pl.BlockSpec((1, d_in), lambda i: (0, 0)),
            pl.BlockSpec((d_in, d_out), lambda i: (0, 0))],
        out_specs=[nar] * nout,
        out_shape=[jax.ShapeDtypeStruct((NPAD, DW), jnp.float32)] * nout,
    )(*aggs, *pblks, deg, b1, w2)


def _tc_final(aggs, pblks, deg, b2):
    d = D_OUT
    nin = d // DW

    def body(*refs):
        ins = refs[:nin]
        ps = refs[nin:2 * nin]
        dg, b_ref, o_ref = refs[2 * nin:]
        denom = dg[...][:, 0:1] + 1.0
        bb = b_ref[...]
        blocks = [(sa[...] + pk[...]) / denom + bb[:, k * DW:(k + 1) * DW]
                  for k, (sa, pk) in enumerate(zip(ins, ps))]
        o_ref[...] = jnp.concatenate(blocks, axis=1)

    nar = pl.BlockSpec((BLK, DW), lambda i: (i, 0))
    return pl.pallas_call(
        body,
        grid=(NPAD // BLK,),
        in_specs=[nar] * (2 * nin) + [
            pl.BlockSpec((BLK, 16), lambda i: (i, 0)),
            pl.BlockSpec((1, d), lambda i: (0, 0))],
        out_specs=pl.BlockSpec((BLK, d), lambda i: (i, 0)),
        out_shape=jax.ShapeDtypeStruct((NPAD, d), jnp.float32),
    )(*aggs, *pblks, deg, b2)


def kernel(x, edge_index, W1, b1, W2, b2):
    f32 = jnp.float32
    src = edge_index[0].astype(jnp.int32)
    dst = edge_index[1].astype(jnp.int32)
    npad_e = EPAD - N_EDGES
    src_p = jnp.concatenate([src, jnp.zeros((npad_e,), jnp.int32)])
    pad_dst = DUMMY + (jnp.arange(npad_e, dtype=jnp.int32) % (NPAD - N_NODES))
    dst_p = jnp.concatenate([dst, pad_dst])
    src_p = src_p.reshape(NT * CPW, CHUNK)
    dst_p = dst_p.reshape(NT * CPW, CHUNK)
    xp = jnp.concatenate([x, jnp.zeros((NPAD - N_NODES, D_IN), f32)])
    zd = jnp.zeros((ROWS_PER_TILE, DW), f32)
    z16 = jnp.zeros((ROWS_PER_TILE, 16), f32)
    ones16 = jnp.ones((CHUNK, 16), f32)
    b1r = b1.reshape(1, D_HID)
    b2r = b2.reshape(1, D_OUT)

    p1b = _tc_matmul_blocks(xp, W1)
    *agg1, deg = _sc_layer(2, True)(*p1b, src_p, dst_p, zd, z16, ones16)
    p2b = _tc_mid(agg1, p1b, deg, b1r, W2)

    agg2 = _sc_layer(1, False)(*p2b, src_p, dst_p, zd, z16, ones16)
    out = _tc_final(agg2, p2b, deg, b2r)
    return out[:N_NODES]


# speedup vs baseline: 2.5538x; 1.0514x over previous
import functools

import jax
import jax.numpy as jnp
from jax import lax
from jax.experimental import pallas as pl
from jax.experimental.pallas import tpu as pltpu
from jax.experimental.pallas import tpu_sc as plsc

N_NODES = 10000
N_EDGES = 320000
D_IN = 128
D_HID = 128
D_OUT = 64
DW = 32

NPAD = 10240
DUMMY = N_NODES
NT = 16
CHUNK = 128
CPW = 160
NBUF = 4
NCH = N_EDGES // CHUNK
LAST_CPW = NCH - 15 * CPW
ROWS_PER_TILE = NPAD // NT


@functools.lru_cache(maxsize=None)
def _sc_layer(npairs, with_deg):
    nblk = 2 * npairs
    mesh = plsc.VectorSubcoreMesh(core_axis_name="c", subcore_axis_name="s",
                                  num_cores=2, num_subcores=16)
    out_t = [jax.ShapeDtypeStruct((NPAD, DW), jnp.float32)] * nblk
    scratch = [
        pltpu.VMEM((CPW, CHUNK), jnp.int32),
        pltpu.VMEM((CPW, CHUNK), jnp.int32),
        [pltpu.VMEM((CHUNK, DW), jnp.float32) for _ in range(NBUF)],
        pltpu.VMEM_SHARED((NPAD, DW), jnp.float32),
        pltpu.VMEM_SHARED((NPAD, DW), jnp.float32),
        [pltpu.SemaphoreType.DMA for _ in range(NBUF)],
    ]
    if with_deg:
        out_t += [jax.ShapeDtypeStruct((NPAD, 16), jnp.float32)]
        scratch += [pltpu.VMEM((CHUNK, 16), jnp.float32),
                    pltpu.VMEM_SHARED((NPAD, 16), jnp.float32)]

    def agg(*refs):
        p_blocks = refs[:nblk]
        ei_hbm, zd_hbm, z16_hbm, ones_hbm = refs[nblk:nblk + 4]
        outs = refs[nblk + 4:2 * nblk + 4]
        rest = refs[2 * nblk + 4:]
        if with_deg:
            deg_out = rest[0]
            (src_v, dst_v, rows, tab, acc, sems, ones_v, dacc) = rest[1:]
        else:
            (src_v, dst_v, rows, tab, acc, sems) = rest
        c = lax.axis_index("c")
        s = lax.axis_index("s")
        rbase = s * ROWS_PER_TILE
        row_slice = pl.ds(rbase, ROWS_PER_TILE)

        @pl.when(s < NT - 1)
        def _():
            pltpu.sync_copy(ei_hbm.at[0, pl.ds(s * CPW, CPW)], src_v)
            pltpu.sync_copy(ei_hbm.at[1, pl.ds(s * CPW, CPW)], dst_v)

        @pl.when(s == NT - 1)
        def _():
            pltpu.sync_copy(ei_hbm.at[0, pl.ds((NT - 1) * CPW, LAST_CPW)],
                            src_v.at[pl.ds(0, LAST_CPW)])
            pltpu.sync_copy(ei_hbm.at[1, pl.ds((NT - 1) * CPW, LAST_CPW)],
                            dst_v.at[pl.ds(0, LAST_CPW)])

        nch_s = lax.select(s < NT - 1, CPW, LAST_CPW)
        if with_deg:
            pltpu.sync_copy(ones_hbm, ones_v)
            pltpu.sync_copy(z16_hbm, dacc.at[row_slice])

        for t in range(npairs):
            @pl.when(c == 0)
            def _():
                pltpu.sync_copy(p_blocks[2 * t].at[row_slice],
                                tab.at[row_slice])

            @pl.when(c == 1)
            def _():
                pltpu.sync_copy(p_blocks[2 * t + 1].at[row_slice],
                                tab.at[row_slice])

            pltpu.sync_copy(zd_hbm, acc.at[row_slice])
            plsc.subcore_barrier()

            for b in range(NBUF):
                pltpu.async_copy(tab.at[src_v.at[b]], rows[b], sems[b])

            count_deg = with_deg and t == 0

            def step(i, carry):
                j = i * NBUF
                for b in range(NBUF):
                    jj = j + b
                    pltpu.make_async_copy(tab.at[src_v.at[jj]], rows[b],
                                          sems[b]).wait()
                    pltpu.sync_copy(rows[b], acc.at[dst_v.at[jj]], add=True)
                    if count_deg:
                        pltpu.sync_copy(ones_v, dacc.at[dst_v.at[jj]],
                                        add=True)

                    @pl.when(jj + NBUF < nch_s)
                    def _():
                        pltpu.async_copy(tab.at[src_v.at[jj + NBUF]],
                                         rows[b], sems[b])
                return carry

            lax.fori_loop(0, nch_s // NBUF, step, 0)
            plsc.subcore_barrier()

            @pl.when(c == 0)
            def _():
                pltpu.sync_copy(acc.at[row_slice], outs[2 * t].at[row_slice])

            @pl.when(c == 1)
            def _():
                pltpu.sync_copy(acc.at[row_slice],
                                outs[2 * t + 1].at[row_slice])

        if with_deg:
            @pl.when(c == 0)
            def _():
                pltpu.sync_copy(dacc.at[row_slice], deg_out.at[row_slice])

    return pl.kernel(
        agg, mesh=mesh, out_type=out_t, scratch_types=scratch,
        compiler_params=pltpu.CompilerParams(use_tc_tiling_on_sc=False))


BLK = 1024


def _tc_matmul_blocks(x, w):
    k, d = w.shape
    nblk = d // DW

    def body(*refs):
        x_ref, w_ref = refs[:2]
        p = jnp.dot(x_ref[...], w_ref[...], preferred_element_type=jnp.float32)
        for i, o_ref in enumerate(refs[2:]):
            o_ref[...] = p[:, i * DW:(i + 1) * DW]

    return pl.pallas_call(
        body,
        grid=(NPAD // BLK,),
        in_specs=[pl.BlockSpec((BLK, k), lambda i: (i, 0)),
                  pl.BlockSpec((k, d), lambda i: (0, 0))],
        out_specs=[pl.BlockSpec((BLK, DW), lambda i: (i, 0))] * nblk,
        out_shape=[jax.ShapeDtypeStruct((NPAD, DW), jnp.float32)] * nblk,
    )(x, w)


def _tc_mid(aggs, pblks, deg, b1, w2):
    d_in, d_out = w2.shape
    nin = d_in // DW
    nout = d_out // DW

    def body(*refs):
        ins = refs[:nin]
        ps = refs[nin:2 * nin]
        dg, b_ref, w_ref = refs[2 * nin:2 * nin + 3]
        outs = refs[2 * nin + 3:]
        denom = dg[...][:, 0:1] + 1.0
        bb = b_ref[...]
        blocks = [(sa[...] + pk[...]) / denom + bb[:, k * DW:(k + 1) * DW]
                  for k, (sa, pk) in enumerate(zip(ins, ps))]
        h = jnp.maximum(jnp.concatenate(blocks, axis=1), 0.0)
        p2 = jnp.dot(h, w_ref[...], preferred_element_type=jnp.float32)
        for k, o_ref in enumerate(outs):
            o_ref[...] = p2[:, k * DW:(k + 1) * DW]

    nar = pl.BlockSpec((BLK, DW), lambda i: (i, 0))
    return pl.pallas_call(
        body,
        grid=(NPAD // BLK,),
        in_specs=[nar] * (2 * nin) + [
            pl.BlockSpec((BLK, 16), lambda i: (i, 0)),
            pl.BlockSpec((1, d_in), lambda i: (0, 0)),
            pl.BlockSpec((d_in, d_out), lambda i: (0, 0))],
        out_specs=[nar] * nout,
        out_shape=[jax.ShapeDtypeStruct((NPAD, DW), jnp.float32)] * nout,
    )(*aggs, *pblks, deg, b1, w2)


def _tc_final(aggs, pblks, deg, b2):
    d = D_OUT
    nin = d // DW

    def body(*refs):
        ins = refs[:nin]
        ps = refs[nin:2 * nin]
        dg, b_ref, o_ref = refs[2 * nin:]
        denom = dg[...][:, 0:1] + 1.0
        bb = b_ref[...]
        blocks = [(sa[...] + pk[...]) / denom + bb[:, k * DW:(k + 1) * DW]
                  for k, (sa, pk) in enumerate(zip(ins, ps))]
        o_ref[...] = jnp.concatenate(blocks, axis=1)

    nar = pl.BlockSpec((BLK, DW), lambda i: (i, 0))
    return pl.pallas_call(
        body,
        grid=(NPAD // BLK,),
        in_specs=[nar] * (2 * nin) + [
            pl.BlockSpec((BLK, 16), lambda i: (i, 0)),
            pl.BlockSpec((1, d), lambda i: (0, 0))],
        out_specs=pl.BlockSpec((BLK, d), lambda i: (i, 0)),
        out_shape=jax.ShapeDtypeStruct((NPAD, d), jnp.float32),
    )(*aggs, *pblks, deg, b2)


def kernel(x, edge_index, W1, b1, W2, b2):
    f32 = jnp.float32
    ei3 = edge_index.astype(jnp.int32).reshape(2, NCH, CHUNK)
    xp = jnp.concatenate([x, jnp.zeros((NPAD - N_NODES, D_IN), f32)])
    zd = jnp.zeros((ROWS_PER_TILE, DW), f32)
    z16 = jnp.zeros((ROWS_PER_TILE, 16), f32)
    ones16 = jnp.ones((CHUNK, 16), f32)
    b1r = b1.reshape(1, D_HID)
    b2r = b2.reshape(1, D_OUT)

    p1b = _tc_matmul_blocks(xp, W1)
    *agg1, deg = _sc_layer(2, True)(*p1b, ei3, zd, z16, ones16)
    p2b = _tc_mid(agg1, p1b, deg, b1r, W2)

    agg2 = _sc_layer(1, False)(*p2b, ei3, zd, z16, ones16)
    out = _tc_final(agg2, p2b, deg, b2r)
    return out[:N_NODES]


# speedup vs baseline: 2.5560x; 1.0009x over previous
import functools

import jax
import jax.numpy as jnp
from jax import lax
from jax.experimental import pallas as pl
from jax.experimental.pallas import tpu as pltpu
from jax.experimental.pallas import tpu_sc as plsc

N_NODES = 10000
N_EDGES = 320000
D_IN = 128
D_HID = 128
D_OUT = 64
DW = 32

NPAD = 10000
NT = 16
CHUNK = 128
CPW = 160
NBUF = 4
NCH = N_EDGES // CHUNK
LAST_CPW = NCH - 15 * CPW
ROWS_PER_TILE = NPAD // NT


@functools.lru_cache(maxsize=None)
def _sc_layer(npairs, with_deg):
    nblk = 2 * npairs
    mesh = plsc.VectorSubcoreMesh(core_axis_name="c", subcore_axis_name="s",
                                  num_cores=2, num_subcores=16)
    out_t = [jax.ShapeDtypeStruct((NPAD, DW), jnp.float32)] * nblk
    scratch = [
        pltpu.VMEM((CPW, CHUNK), jnp.int32),
        pltpu.VMEM((CPW, CHUNK), jnp.int32),
        [pltpu.VMEM((CHUNK, DW), jnp.float32) for _ in range(NBUF)],
        pltpu.VMEM_SHARED((NPAD, DW), jnp.float32),
        pltpu.VMEM_SHARED((NPAD, DW), jnp.float32),
        [pltpu.SemaphoreType.DMA for _ in range(NBUF)],
    ]
    if with_deg:
        out_t += [jax.ShapeDtypeStruct((NPAD, 16), jnp.float32)]
        scratch += [pltpu.VMEM((CHUNK, 16), jnp.float32),
                    pltpu.VMEM_SHARED((NPAD, 16), jnp.float32)]

    def agg(*refs):
        p_blocks = refs[:nblk]
        ei_hbm, zd_hbm, z16_hbm, ones_hbm = refs[nblk:nblk + 4]
        outs = refs[nblk + 4:2 * nblk + 4]
        rest = refs[2 * nblk + 4:]
        if with_deg:
            deg_out = rest[0]
            (src_v, dst_v, rows, tab, acc, sems, ones_v, dacc) = rest[1:]
        else:
            (src_v, dst_v, rows, tab, acc, sems) = rest
        c = lax.axis_index("c")
        s = lax.axis_index("s")
        rbase = s * ROWS_PER_TILE
        row_slice = pl.ds(rbase, ROWS_PER_TILE)

        @pl.when(s < NT - 1)
        def _():
            pltpu.sync_copy(ei_hbm.at[0, pl.ds(s * CPW, CPW)], src_v)
            pltpu.sync_copy(ei_hbm.at[1, pl.ds(s * CPW, CPW)], dst_v)

        @pl.when(s == NT - 1)
        def _():
            pltpu.sync_copy(ei_hbm.at[0, pl.ds((NT - 1) * CPW, LAST_CPW)],
                            src_v.at[pl.ds(0, LAST_CPW)])
            pltpu.sync_copy(ei_hbm.at[1, pl.ds((NT - 1) * CPW, LAST_CPW)],
                            dst_v.at[pl.ds(0, LAST_CPW)])

        nch_s = lax.select(s < NT - 1, CPW, LAST_CPW)
        if with_deg:
            pltpu.sync_copy(ones_hbm, ones_v)
            pltpu.sync_copy(z16_hbm, dacc.at[row_slice])

        for t in range(npairs):
            @pl.when(c == 0)
            def _():
                pltpu.sync_copy(p_blocks[2 * t].at[row_slice],
                                tab.at[row_slice])

            @pl.when(c == 1)
            def _():
                pltpu.sync_copy(p_blocks[2 * t + 1].at[row_slice],
                                tab.at[row_slice])

            pltpu.sync_copy(zd_hbm, acc.at[row_slice])
            plsc.subcore_barrier()

            for b in range(NBUF):
                pltpu.async_copy(tab.at[src_v.at[b]], rows[b], sems[b])

            count_deg = with_deg and t == 0

            def step(i, carry):
                j = i * NBUF
                for b in range(NBUF):
                    jj = j + b
                    pltpu.make_async_copy(tab.at[src_v.at[jj]], rows[b],
                                          sems[b]).wait()
                    pltpu.sync_copy(rows[b], acc.at[dst_v.at[jj]], add=True)
                    if count_deg:
                        pltpu.sync_copy(ones_v, dacc.at[dst_v.at[jj]],
                                        add=True)

                    @pl.when(jj + NBUF < nch_s)
                    def _():
                        pltpu.async_copy(tab.at[src_v.at[jj + NBUF]],
                                         rows[b], sems[b])
                return carry

            lax.fori_loop(0, nch_s // NBUF, step, 0)
            plsc.subcore_barrier()

            @pl.when(c == 0)
            def _():
                pltpu.sync_copy(acc.at[row_slice], outs[2 * t].at[row_slice])

            @pl.when(c == 1)
            def _():
                pltpu.sync_copy(acc.at[row_slice],
                                outs[2 * t + 1].at[row_slice])

        if with_deg:
            @pl.when(c == 0)
            def _():
                pltpu.sync_copy(dacc.at[row_slice], deg_out.at[row_slice])

    return pl.kernel(
        agg, mesh=mesh, out_type=out_t, scratch_types=scratch,
        compiler_params=pltpu.CompilerParams(use_tc_tiling_on_sc=False))


BLK = 1000


def _tc_matmul_blocks(x, w):
    k, d = w.shape
    nblk = d // DW

    def body(*refs):
        x_ref, w_ref = refs[:2]
        p = jnp.dot(x_ref[...], w_ref[...], preferred_element_type=jnp.float32)
        for i, o_ref in enumerate(refs[2:]):
            o_ref[...] = p[:, i * DW:(i + 1) * DW]

    return pl.pallas_call(
        body,
        grid=(NPAD // BLK,),
        in_specs=[pl.BlockSpec((BLK, k), lambda i: (i, 0)),
                  pl.BlockSpec((k, d), lambda i: (0, 0))],
        out_specs=[pl.BlockSpec((BLK, DW), lambda i: (i, 0))] * nblk,
        out_shape=[jax.ShapeDtypeStruct((NPAD, DW), jnp.float32)] * nblk,
    )(x, w)


def _tc_mid(aggs, pblks, deg, b1, w2):
    d_in, d_out = w2.shape
    nin = d_in // DW
    nout = d_out // DW

    def body(*refs):
        ins = refs[:nin]
        ps = refs[nin:2 * nin]
        dg, b_ref, w_ref = refs[2 * nin:2 * nin + 3]
        outs = refs[2 * nin + 3:]
        denom = dg[...][:, 0:1] + 1.0
        bb = b_ref[...]
        blocks = [(sa[...] + pk[...]) / denom + bb[:, k * DW:(k + 1) * DW]
                  for k, (sa, pk) in enumerate(zip(ins, ps))]
        h = jnp.maximum(jnp.concatenate(blocks, axis=1), 0.0)
        p2 = jnp.dot(h, w_ref[...], preferred_element_type=jnp.float32)
        for k, o_ref in enumerate(outs):
            o_ref[...] = p2[:, k * DW:(k + 1) * DW]

    nar = pl.BlockSpec((BLK, DW), lambda i: (i, 0))
    return pl.pallas_call(
        body,
        grid=(NPAD // BLK,),
        in_specs=[nar] * (2 * nin) + [
            pl.BlockSpec((BLK, 16), lambda i: (i, 0)),
            pl.BlockSpec((1, d_in), lambda i: (0, 0)),
            pl.BlockSpec((d_in, d_out), lambda i: (0, 0))],
        out_specs=[nar] * nout,
        out_shape=[jax.ShapeDtypeStruct((NPAD, DW), jnp.float32)] * nout,
    )(*aggs, *pblks, deg, b1, w2)


def _tc_final(aggs, pblks, deg, b2):
    d = D_OUT
    nin = d // DW

    def body(*refs):
        ins = refs[:nin]
        ps = refs[nin:2 * nin]
        dg, b_ref, o_ref = refs[2 * nin:]
        denom = dg[...][:, 0:1] + 1.0
        bb = b_ref[...]
        blocks = [(sa[...] + pk[...]) / denom + bb[:, k * DW:(k + 1) * DW]
                  for k, (sa, pk) in enumerate(zip(ins, ps))]
        o_ref[...] = jnp.concatenate(blocks, axis=1)

    nar = pl.BlockSpec((BLK, DW), lambda i: (i, 0))
    return pl.pallas_call(
        body,
        grid=(NPAD // BLK,),
        in_specs=[nar] * (2 * nin) + [
            pl.BlockSpec((BLK, 16), lambda i: (i, 0)),
            pl.BlockSpec((1, d), lambda i: (0, 0))],
        out_specs=pl.BlockSpec((BLK, d), lambda i: (i, 0)),
        out_shape=jax.ShapeDtypeStruct((NPAD, d), jnp.float32),
    )(*aggs, *pblks, deg, b2)


def kernel(x, edge_index, W1, b1, W2, b2):
    f32 = jnp.float32
    ei3 = edge_index.astype(jnp.int32).reshape(2, NCH, CHUNK)
    zd = jnp.zeros((ROWS_PER_TILE, DW), f32)
    z16 = jnp.zeros((ROWS_PER_TILE, 16), f32)
    ones16 = jnp.ones((CHUNK, 16), f32)
    b1r = b1.reshape(1, D_HID)
    b2r = b2.reshape(1, D_OUT)

    p1b = _tc_matmul_blocks(x, W1)
    *agg1, deg = _sc_layer(2, True)(*p1b, ei3, zd, z16, ones16)
    p2b = _tc_mid(agg1, p1b, deg, b1r, W2)

    agg2 = _sc_layer(1, False)(*p2b, ei3, zd, z16, ones16)
    return _tc_final(agg2, p2b, deg, b2r)
